# baseline (speedup 1.0000x reference)
import functools

import jax
import jax.numpy as jnp
from jax import lax
from jax.experimental import pallas as pl
from jax.experimental.pallas import tpu as pltpu
from jax.experimental.pallas import tpu_sc as plsc

NC, NS, L = 2, 16, 16
NW = NC * NS
B, S, D = 4, 8192, 768
POS_PER_W = S // NW
C = 32
H = C // 2
NCHUNK = POS_PER_W // C
NSTEP = NCHUNK * B
DV = D // L
NBUF = 4
NWPE = 1


def _emb_body(ids_hbm, wte_hbm, wpe_hbm, out_hbm, idx_v, wpe_v, rows_v,
              gsem, wsem, psem):
    wid = lax.axis_index("s") * NC + lax.axis_index("c")
    p_base = wid * POS_PER_W

    for b in range(B):
        pltpu.sync_copy(
            ids_hbm.at[pl.ds(b * S + p_base, POS_PER_W)],
            idx_v.at[pl.ds(b * POS_PER_W, POS_PER_W)],
        )

    def start_wpe(pc, wsel):
        return pltpu.async_copy(
            wpe_hbm.at[pl.ds(p_base + pc * C, C)], wpe_v.at[wsel], psem.at[wsel]
        )

    def gather_half(n, h):
        pc, b, buf = n // B, n % B, n % NBUF
        return pltpu.make_async_copy(
            wte_hbm.at[idx_v.at[pl.ds(b * POS_PER_W + pc * C + h * H, H)]],
            rows_v.at[buf, pl.ds(h * H, H)],
            gsem.at[buf, h],
        )

    def write_half(n, h):
        pc, b, buf = n // B, n % B, n % NBUF
        return pltpu.make_async_copy(
            rows_v.at[buf, pl.ds(h * H, H)],
            out_hbm.at[pl.ds(b * S + p_base + pc * C + h * H, H)],
            wsem.at[buf, h],
        )

    def start_gather(n):
        for h in range(2):
            gather_half(n, h).start()

    LEAD = NBUF - 1
    wpe_pending = [start_wpe(pc, w) for pc, w in zip(range(NWPE), range(NWPE))]
    for n in range(LEAD):
        start_gather(n)

    for n in range(NSTEP):
        pc, b, buf, wsel = n // B, n % B, n % NBUF, (n // B) % NWPE
        if b == 0:
            wpe_pending[wsel].wait()

        def half_body(h, _):
            pltpu.make_async_copy(
                wte_hbm.at[idx_v.at[pl.ds(b * POS_PER_W + pc * C + h * H, H)]],
                rows_v.at[buf, pl.ds(h * H, H)],
                gsem.at[buf, h],
            ).wait()

            def add_body(r, _):
                for q in range(2):
                    j0 = q * (DV // 2)
                    sums = [
                        rows_v[buf, r, pl.ds((j0 + j) * L, L)]
                        + wpe_v[wsel, r, pl.ds((j0 + j) * L, L)]
                        for j in range(DV // 2)
                    ]
                    for j in range(DV // 2):
                        rows_v[buf, r, pl.ds((j0 + j) * L, L)] = sums[j]
                return 0

            lax.fori_loop(h * H, (h + 1) * H, add_body, 0)
            pltpu.make_async_copy(
                rows_v.at[buf, pl.ds(h * H, H)],
                out_hbm.at[pl.ds(b * S + p_base + pc * C + h * H, H)],
                wsem.at[buf, h],
            ).start()
            return 0

        lax.fori_loop(0, 2, half_body, 0)

        if b == B - 1 and pc + NWPE < NCHUNK:
            wpe_pending[wsel] = start_wpe(pc + NWPE, wsel)
        nxt = n + LEAD
        if nxt < NSTEP:
            prev = nxt - NBUF
            if prev >= 0:
                for h in range(2):
                    write_half(prev, h).wait()
            start_gather(nxt)

    for prev in range(NSTEP - NBUF, NSTEP):
        for h in range(2):
            write_half(prev, h).wait()


@jax.jit
def _emb(ids_flat, wte, wpe):
    mesh = plsc.VectorSubcoreMesh(core_axis_name="c", subcore_axis_name="s")
    f = pl.kernel(
        _emb_body,
        out_type=jax.ShapeDtypeStruct((B * S, D), jnp.float32),
        mesh=mesh,
        scratch_types=[
            pltpu.VMEM((B * POS_PER_W,), jnp.int32),
            pltpu.VMEM((NWPE, C, D), jnp.float32),
            pltpu.VMEM((NBUF, C, D), jnp.float32),
            pltpu.SemaphoreType.DMA((NBUF, 2)),
            pltpu.SemaphoreType.DMA((NBUF, 2)),
            pltpu.SemaphoreType.DMA((NWPE,)),
        ],
    )
    return f(ids_flat, wte, wpe)


def kernel(input_ids, wte, wpe):
    ids_flat = input_ids.reshape(-1)
    out = _emb(ids_flat, wte, wpe)
    return out.reshape(input_ids.shape[0], S, D)


# speedup vs baseline: 1.1507x; 1.1507x over previous
import functools

import jax
import jax.numpy as jnp
from jax import lax
from jax.experimental import pallas as pl
from jax.experimental.pallas import tpu as pltpu
from jax.experimental.pallas import tpu_sc as plsc

NC, NS, L = 2, 16, 16
NW = NC * NS
B, S, D = 4, 8192, 768
POS_PER_W = S // NW
C = 32
NSPLIT = 4
H = C // NSPLIT
NCHUNK = POS_PER_W // C
NSTEP = NCHUNK * B
DV = D // L
NBUF = 3
NWPE = 2


def _emb_body(ids_hbm, wte_hbm, wpe_hbm, out_hbm, idx_v, wpe_v, rows_v,
              gsem, wsem, psem):
    wid = lax.axis_index("s") * NC + lax.axis_index("c")
    p_base = wid * POS_PER_W

    for b in range(B):
        pltpu.sync_copy(
            ids_hbm.at[pl.ds(b * S + p_base, POS_PER_W)],
            idx_v.at[pl.ds(b * POS_PER_W, POS_PER_W)],
        )

    def start_wpe(pc, wsel):
        return pltpu.async_copy(
            wpe_hbm.at[pl.ds(p_base + pc * C, C)], wpe_v.at[wsel], psem.at[wsel]
        )

    def gather_half(n, h):
        pc, b, buf = n // B, n % B, n % NBUF
        return pltpu.make_async_copy(
            wte_hbm.at[idx_v.at[pl.ds(b * POS_PER_W + pc * C + h * H, H)]],
            rows_v.at[buf, pl.ds(h * H, H)],
            gsem.at[buf, h],
        )

    def write_half(n, h):
        pc, b, buf = n // B, n % B, n % NBUF
        return pltpu.make_async_copy(
            rows_v.at[buf, pl.ds(h * H, H)],
            out_hbm.at[pl.ds(b * S + p_base + pc * C + h * H, H)],
            wsem.at[buf, h],
        )

    def start_gather(n):
        for h in range(NSPLIT):
            gather_half(n, h).start()

    LEAD = NBUF - 1
    wpe_pending = [start_wpe(pc, w) for pc, w in zip(range(NWPE), range(NWPE))]
    for n in range(LEAD):
        start_gather(n)

    for n in range(NSTEP):
        pc, b, buf, wsel = n // B, n % B, n % NBUF, (n // B) % NWPE
        if b == 0:
            wpe_pending[wsel].wait()

        def half_body(h, _):
            pltpu.make_async_copy(
                wte_hbm.at[idx_v.at[pl.ds(b * POS_PER_W + pc * C + h * H, H)]],
                rows_v.at[buf, pl.ds(h * H, H)],
                gsem.at[buf, h],
            ).wait()

            def add_body(r, _):
                for q in range(2):
                    j0 = q * (DV // 2)
                    sums = [
                        rows_v[buf, r, pl.ds((j0 + j) * L, L)]
                        + wpe_v[wsel, r, pl.ds((j0 + j) * L, L)]
                        for j in range(DV // 2)
                    ]
                    for j in range(DV // 2):
                        rows_v[buf, r, pl.ds((j0 + j) * L, L)] = sums[j]
                return 0

            lax.fori_loop(h * H, (h + 1) * H, add_body, 0)
            pltpu.make_async_copy(
                rows_v.at[buf, pl.ds(h * H, H)],
                out_hbm.at[pl.ds(b * S + p_base + pc * C + h * H, H)],
                wsem.at[buf, h],
            ).start()
            return 0

        lax.fori_loop(0, NSPLIT, half_body, 0)

        if b == B - 1 and pc + NWPE < NCHUNK:
            wpe_pending[wsel] = start_wpe(pc + NWPE, wsel)
        nxt = n + LEAD
        if nxt < NSTEP:
            prev = nxt - NBUF
            if prev >= 0:
                for h in range(NSPLIT):
                    write_half(prev, h).wait()
            start_gather(nxt)

    for prev in range(NSTEP - NBUF, NSTEP):
        for h in range(NSPLIT):
            write_half(prev, h).wait()


@jax.jit
def _emb(ids_flat, wte, wpe):
    mesh = plsc.VectorSubcoreMesh(core_axis_name="c", subcore_axis_name="s")
    f = pl.kernel(
        _emb_body,
        out_type=jax.ShapeDtypeStruct((B * S, D), jnp.float32),
        mesh=mesh,
        scratch_types=[
            pltpu.VMEM((B * POS_PER_W,), jnp.int32),
            pltpu.VMEM((NWPE, C, D), jnp.float32),
            pltpu.VMEM((NBUF, C, D), jnp.float32),
            pltpu.SemaphoreType.DMA((NBUF, NSPLIT)),
            pltpu.SemaphoreType.DMA((NBUF, NSPLIT)),
            pltpu.SemaphoreType.DMA((NWPE,)),
        ],
    )
    return f(ids_flat, wte, wpe)


def kernel(input_ids, wte, wpe):
    ids_flat = input_ids.reshape(-1)
    out = _emb(ids_flat, wte, wpe)
    return out.reshape(input_ids.shape[0], S, D)
